# baseline (speedup 1.0000x reference)
import functools

import jax
import jax.numpy as jnp
from jax.experimental import pallas as pl
from jax.experimental.pallas import tpu as pltpu

_H = 1024
_F = 512
_E = 8
_K = 2


def _router_shared_body(x_ref, gwt_ref, swg_ref, swu_ref, swd_ref,
                        comb_ref, shared_ref):
    x = x_ref[...]
    logits = jnp.dot(x, gwt_ref[...], preferred_element_type=jnp.float32)
    scores = jax.nn.sigmoid(logits)
    iota = jax.lax.broadcasted_iota(jnp.int32, scores.shape, 1)
    m1 = jnp.max(scores, axis=-1, keepdims=True)
    i1 = jnp.min(jnp.where(scores == m1, iota, _E), axis=-1, keepdims=True)
    masked = jnp.where(iota == i1, -jnp.inf, scores)
    m2 = jnp.max(masked, axis=-1, keepdims=True)
    i2 = jnp.min(jnp.where(masked == m2, iota, _E), axis=-1, keepdims=True)
    denom = m1 + m2 + 1e-20
    comb = jnp.where(iota == i1, m1, 0.0) + jnp.where(iota == i2, m2, 0.0)
    comb_ref[...] = comb / denom

    h1 = jnp.dot(x, swg_ref[...], preferred_element_type=jnp.float32)
    g = h1 * jax.nn.sigmoid(h1)
    u = jnp.dot(x, swu_ref[...], preferred_element_type=jnp.float32)
    shared_ref[...] = jnp.dot(g * u, swd_ref[...],
                              preferred_element_type=jnp.float32)


def _experts_body(comb_ref, x_ref, wg_ref, wu_ref, wd_ref, shared_ref,
                  out_ref):
    e = pl.program_id(1)
    x = x_ref[...]
    h1 = jnp.dot(x, wg_ref[0], preferred_element_type=jnp.float32)
    g = h1 * jax.nn.sigmoid(h1)
    u = jnp.dot(x, wu_ref[0], preferred_element_type=jnp.float32)
    o = jnp.dot(g * u, wd_ref[0], preferred_element_type=jnp.float32)
    w = comb_ref[0, 0, :]
    contrib = o * w[:, None]

    @pl.when(e == 0)
    def _init():
        out_ref[...] = shared_ref[...] + contrib

    @pl.when(e > 0)
    def _acc():
        out_ref[...] += contrib


def kernel(hidden_states, gate_weight, w_gate, w_up, w_down,
           sw_gate, sw_up, sw_down):
    b, s, h = hidden_states.shape
    t = b * s
    x = hidden_states.reshape(t, h)

    comb, shared = pl.pallas_call(
        _router_shared_body,
        out_shape=(
            jax.ShapeDtypeStruct((t, _E), jnp.float32),
            jax.ShapeDtypeStruct((t, h), jnp.float32),
        ),
    )(x, gate_weight.T, sw_gate, sw_up, sw_down)

    comb3 = comb.T.reshape(_E, 1, t)

    tt = 256
    grid = (t // tt, _E)
    y = pl.pallas_call(
        _experts_body,
        grid=grid,
        in_specs=[
            pl.BlockSpec((1, 1, tt), lambda i, e: (e, 0, i)),
            pl.BlockSpec((tt, h), lambda i, e: (i, 0)),
            pl.BlockSpec((1, h, _F), lambda i, e: (e, 0, 0)),
            pl.BlockSpec((1, h, _F), lambda i, e: (e, 0, 0)),
            pl.BlockSpec((1, _F, h), lambda i, e: (e, 0, 0)),
            pl.BlockSpec((tt, h), lambda i, e: (i, 0)),
        ],
        out_specs=pl.BlockSpec((tt, h), lambda i, e: (i, 0)),
        out_shape=jax.ShapeDtypeStruct((t, h), jnp.float32),
    )(comb3, x, w_gate, w_up, w_down, shared)

    return y.reshape(b, s, h)


# speedup vs baseline: 1.0119x; 1.0119x over previous
import functools

import jax
import jax.numpy as jnp
from jax.experimental import pallas as pl
from jax.experimental.pallas import tpu as pltpu

_H = 1024
_F = 512
_E = 8
_K = 2


def _router_shared_body(x_ref, gwt_ref, swg_ref, swu_ref, swd_ref,
                        comb_ref, shared_ref):
    x = x_ref[...]
    logits = jnp.dot(x, gwt_ref[...], preferred_element_type=jnp.float32)
    scores = jax.nn.sigmoid(logits)
    iota = jax.lax.broadcasted_iota(jnp.int32, scores.shape, 1)
    m1 = jnp.max(scores, axis=-1, keepdims=True)
    i1 = jnp.min(jnp.where(scores == m1, iota, _E), axis=-1, keepdims=True)
    masked = jnp.where(iota == i1, -jnp.inf, scores)
    m2 = jnp.max(masked, axis=-1, keepdims=True)
    i2 = jnp.min(jnp.where(masked == m2, iota, _E), axis=-1, keepdims=True)
    denom = m1 + m2 + 1e-20
    comb = jnp.where(iota == i1, m1, 0.0) + jnp.where(iota == i2, m2, 0.0)
    comb_ref[...] = comb / denom

    xb = x.astype(jnp.bfloat16)
    h1 = jnp.dot(xb, swg_ref[...], preferred_element_type=jnp.float32)
    g = h1 * jax.nn.sigmoid(h1)
    u = jnp.dot(xb, swu_ref[...], preferred_element_type=jnp.float32)
    shared_ref[...] = jnp.dot((g * u).astype(jnp.bfloat16), swd_ref[...],
                              preferred_element_type=jnp.float32)


def _experts_body(comb_ref, x_ref, wg_ref, wu_ref, wd_ref, shared_ref,
                  out_ref):
    e = pl.program_id(1)
    x = x_ref[...]
    h1 = jnp.dot(x, wg_ref[0], preferred_element_type=jnp.float32)
    g = h1 * jax.nn.sigmoid(h1)
    u = jnp.dot(x, wu_ref[0], preferred_element_type=jnp.float32)
    o = jnp.dot((g * u).astype(jnp.bfloat16), wd_ref[0],
                preferred_element_type=jnp.float32)
    w = comb_ref[0, 0, :]
    contrib = o * w[:, None]

    @pl.when(e == 0)
    def _init():
        out_ref[...] = shared_ref[...] + contrib

    @pl.when(e > 0)
    def _acc():
        out_ref[...] += contrib


def kernel(hidden_states, gate_weight, w_gate, w_up, w_down,
           sw_gate, sw_up, sw_down):
    b, s, h = hidden_states.shape
    t = b * s
    x = hidden_states.reshape(t, h)

    bf = jnp.bfloat16
    comb, shared = pl.pallas_call(
        _router_shared_body,
        out_shape=(
            jax.ShapeDtypeStruct((t, _E), jnp.float32),
            jax.ShapeDtypeStruct((t, h), jnp.float32),
        ),
    )(x, gate_weight.T, sw_gate.astype(bf), sw_up.astype(bf),
      sw_down.astype(bf))

    comb3 = comb.T.reshape(_E, 1, t)
    xb = x.astype(bf)

    tt = 256
    grid = (t // tt, _E)
    y = pl.pallas_call(
        _experts_body,
        grid=grid,
        in_specs=[
            pl.BlockSpec((1, 1, tt), lambda i, e: (e, 0, i)),
            pl.BlockSpec((tt, h), lambda i, e: (i, 0)),
            pl.BlockSpec((1, h, _F), lambda i, e: (e, 0, 0)),
            pl.BlockSpec((1, h, _F), lambda i, e: (e, 0, 0)),
            pl.BlockSpec((1, _F, h), lambda i, e: (e, 0, 0)),
            pl.BlockSpec((tt, h), lambda i, e: (i, 0)),
        ],
        out_specs=pl.BlockSpec((tt, h), lambda i, e: (i, 0)),
        out_shape=jax.ShapeDtypeStruct((t, h), jnp.float32),
    )(comb3, xb, w_gate.astype(bf), w_up.astype(bf), w_down.astype(bf),
      shared)

    return y.reshape(b, s, h)


# speedup vs baseline: 1.3016x; 1.2862x over previous
import functools

import jax
import jax.numpy as jnp
from jax.experimental import pallas as pl
from jax.experimental.pallas import tpu as pltpu

_H = 1024
_F = 512
_E = 8
_K = 2


def _router_shared_body(x_ref, gwt_ref, swg_ref, swu_ref, swd_ref,
                        comb_ref, shared_ref):
    x = x_ref[...]
    logits = jnp.dot(x, gwt_ref[...], preferred_element_type=jnp.float32)
    scores = jax.nn.sigmoid(logits)
    iota = jax.lax.broadcasted_iota(jnp.int32, scores.shape, 1)
    m1 = jnp.max(scores, axis=-1, keepdims=True)
    i1 = jnp.min(jnp.where(scores == m1, iota, _E), axis=-1, keepdims=True)
    masked = jnp.where(iota == i1, -jnp.inf, scores)
    m2 = jnp.max(masked, axis=-1, keepdims=True)
    i2 = jnp.min(jnp.where(masked == m2, iota, _E), axis=-1, keepdims=True)
    denom = m1 + m2 + 1e-20
    comb = jnp.where(iota == i1, m1, 0.0) + jnp.where(iota == i2, m2, 0.0)
    comb_ref[...] = comb / denom

    xb = x.astype(jnp.bfloat16)
    h1 = jnp.dot(xb, swg_ref[...], preferred_element_type=jnp.float32)
    g = h1 * jax.nn.sigmoid(h1)
    u = jnp.dot(xb, swu_ref[...], preferred_element_type=jnp.float32)
    shared_ref[...] = jnp.dot((g * u).astype(jnp.bfloat16), swd_ref[...],
                              preferred_element_type=jnp.float32)


def _experts_body(comb_ref, x_ref, wg_ref, wu_ref, wd_ref, shared_ref,
                  out_ref):
    x = x_ref[...]
    acc = shared_ref[...]
    for e in range(_E):
        h1 = jnp.dot(x, wg_ref[e], preferred_element_type=jnp.float32)
        g = h1 * jax.nn.sigmoid(h1)
        u = jnp.dot(x, wu_ref[e], preferred_element_type=jnp.float32)
        o = jnp.dot((g * u).astype(jnp.bfloat16), wd_ref[e],
                    preferred_element_type=jnp.float32)
        acc = acc + o * comb_ref[:, e][:, None]
    out_ref[...] = acc


def kernel(hidden_states, gate_weight, w_gate, w_up, w_down,
           sw_gate, sw_up, sw_down):
    b, s, h = hidden_states.shape
    t = b * s
    x = hidden_states.reshape(t, h)

    bf = jnp.bfloat16
    comb, shared = pl.pallas_call(
        _router_shared_body,
        out_shape=(
            jax.ShapeDtypeStruct((t, _E), jnp.float32),
            jax.ShapeDtypeStruct((t, h), jnp.float32),
        ),
    )(x, gate_weight.T, sw_gate.astype(bf), sw_up.astype(bf),
      sw_down.astype(bf))

    xb = x.astype(bf)

    tt = 256
    grid = (t // tt,)
    y = pl.pallas_call(
        _experts_body,
        grid=grid,
        in_specs=[
            pl.BlockSpec((tt, _E), lambda i: (i, 0)),
            pl.BlockSpec((tt, h), lambda i: (i, 0)),
            pl.BlockSpec((_E, h, _F), lambda i: (0, 0, 0)),
            pl.BlockSpec((_E, h, _F), lambda i: (0, 0, 0)),
            pl.BlockSpec((_E, _F, h), lambda i: (0, 0, 0)),
            pl.BlockSpec((tt, h), lambda i: (i, 0)),
        ],
        out_specs=pl.BlockSpec((tt, h), lambda i: (i, 0)),
        out_shape=jax.ShapeDtypeStruct((t, h), jnp.float32),
    )(comb, xb, w_gate.astype(bf), w_up.astype(bf), w_down.astype(bf),
      shared)

    return y.reshape(b, s, h)
